# Initial kernel scaffold; baseline (speedup 1.0000x reference)
import functools

import jax
import jax.numpy as jnp
from jax import lax
from jax.experimental import pallas as pl
from jax.experimental.pallas import tpu as pltpu
from jax.experimental.pallas import tpu_sc as plsc

D = 32
NC, NS = 2, 16
NW = NC * NS
K = 13
CH = K * 128


@functools.lru_cache(maxsize=None)
def _emb_kernel(N):
    per_w = N // NW
    n_rows = per_w // 128
    steps = n_rows // K
    mesh = plsc.VectorSubcoreMesh(core_axis_name="c", subcore_axis_name="s")

    @functools.partial(
        pl.kernel,
        out_type=jax.ShapeDtypeStruct((N, D), jnp.float32),
        mesh=mesh,
        scratch_types=[
            pltpu.VMEM((K, 128), jnp.int32),
            pltpu.VMEM((CH, D), jnp.float32),
            pltpu.SemaphoreType.DMA,
        ],
    )
    def body(table_hbm, idx_hbm, out_hbm, idx_v, rows_v, sem):
        wid = lax.axis_index("s") * NC + lax.axis_index("c")
        row0 = wid * n_rows

        def step(i, carry):
            r = row0 + i * K
            pltpu.sync_copy(idx_hbm.at[pl.ds(r, K)], idx_v)
            copies = [
                pltpu.async_copy(
                    table_hbm.at[idx_v.at[j]],
                    rows_v.at[pl.ds(j * 128, 128)],
                    sem,
                )
                for j in range(K)
            ]
            for c in copies:
                c.wait()
            pltpu.sync_copy(rows_v, out_hbm.at[pl.ds(r * 128, CH)])
            return carry

        lax.fori_loop(0, steps, step, 0)

    return body


def kernel(input, table):
    s0, s1, s2 = input.shape
    N = s0 * s1 * s2
    idx2d = input.reshape(N // 128, 128).astype(jnp.int32)
    out = _emb_kernel(N)(table, idx2d)
    return out.reshape(s0, s1, s2, D)


# speedup vs baseline: 2.6129x; 2.6129x over previous
import functools

import jax
import jax.numpy as jnp
from jax import lax
from jax.experimental import pallas as pl
from jax.experimental.pallas import tpu as pltpu
from jax.experimental.pallas import tpu_sc as plsc

D = 32
NC, NS = 2, 16
NW = NC * NS
K = 8
CH = K * 128


@functools.lru_cache(maxsize=None)
def _emb_kernel(N):
    per_w = N // NW
    n_rows = per_w // 128
    steps = n_rows // K
    mesh = plsc.VectorSubcoreMesh(core_axis_name="c", subcore_axis_name="s")

    @functools.partial(
        pl.kernel,
        out_type=jax.ShapeDtypeStruct((N, D), jnp.float32),
        mesh=mesh,
        scratch_types=[
            pltpu.VMEM((K, 128), jnp.int32),
            pltpu.VMEM((CH, D), jnp.float32),
            pltpu.SemaphoreType.DMA,
        ],
        compiler_params=pltpu.CompilerParams(use_tc_tiling_on_sc=False),
    )
    def body(table_hbm, idx_hbm, out_hbm, idx_v, rows_v, sem):
        wid = lax.axis_index("s") * NC + lax.axis_index("c")
        row0 = wid * n_rows

        def step(i, carry):
            r = row0 + i * K
            pltpu.sync_copy(idx_hbm.at[pl.ds(r, K)], idx_v)
            copies = [
                pltpu.async_copy(
                    table_hbm.at[idx_v.at[j]],
                    rows_v.at[pl.ds(j * 128, 128)],
                    sem,
                )
                for j in range(K)
            ]
            for c in copies:
                c.wait()
            pltpu.sync_copy(rows_v, out_hbm.at[pl.ds(r * 128, CH)])
            return carry

        lax.fori_loop(0, steps, step, 0)

    return body


def kernel(input, table):
    s0, s1, s2 = input.shape
    N = s0 * s1 * s2
    idx2d = input.reshape(N // 128, 128).astype(jnp.int32)
    out = _emb_kernel(N)(table, idx2d)
    return out.reshape(s0, s1, s2, D)


# speedup vs baseline: 2.7241x; 1.0426x over previous
import functools

import jax
import jax.numpy as jnp
from jax import lax
from jax.experimental import pallas as pl
from jax.experimental.pallas import tpu as pltpu
from jax.experimental.pallas import tpu_sc as plsc

D = 32
NC, NS = 2, 16
NW = NC * NS
K = 8
CH = K * 128


@functools.lru_cache(maxsize=None)
def _emb_kernel(N):
    per_w = N // NW
    n_rows = per_w // 128
    steps = n_rows // K
    assert steps * K == n_rows and steps >= 3
    mesh = plsc.VectorSubcoreMesh(core_axis_name="c", subcore_axis_name="s")

    @functools.partial(
        pl.kernel,
        out_type=jax.ShapeDtypeStruct((N, D), jnp.float32),
        mesh=mesh,
        scratch_types=[
            pltpu.VMEM((2, K, 128), jnp.int32),
            pltpu.VMEM((2, CH, D), jnp.float32),
            pltpu.SemaphoreType.DMA((2,)),
            pltpu.SemaphoreType.DMA((2,)),
            pltpu.SemaphoreType.DMA((2,)),
        ],
        compiler_params=pltpu.CompilerParams(use_tc_tiling_on_sc=False),
    )
    def body(table_hbm, idx_hbm, out_hbm, idx_v, rows_v, sem_i, sem_g, sem_o):
        wid = lax.axis_index("s") * NC + lax.axis_index("c")
        row0 = wid * n_rows

        def fire_idx(slot, t):
            pltpu.async_copy(
                idx_hbm.at[pl.ds(row0 + t * K, K)], idx_v.at[slot], sem_i.at[slot]
            )

        def wait_idx(slot):
            pltpu.make_async_copy(
                idx_hbm.at[pl.ds(0, K)], idx_v.at[slot], sem_i.at[slot]
            ).wait()

        def fire_gathers(slot):
            for j in range(K):
                pltpu.async_copy(
                    table_hbm.at[idx_v.at[slot].at[j]],
                    rows_v.at[slot].at[pl.ds(j * 128, 128)],
                    sem_g.at[slot],
                )

        def wait_gathers(slot):
            pltpu.make_async_copy(
                table_hbm.at[pl.ds(0, CH)], rows_v.at[slot], sem_g.at[slot]
            ).wait()

        def fire_out(slot, t):
            pltpu.async_copy(
                rows_v.at[slot],
                out_hbm.at[pl.ds((row0 + t * K) * 128, CH)],
                sem_o.at[slot],
            )

        def wait_out(slot):
            pltpu.make_async_copy(
                rows_v.at[slot], out_hbm.at[pl.ds(0, CH)], sem_o.at[slot]
            ).wait()

        fire_idx(0, 0)
        wait_idx(0)
        fire_gathers(0)
        fire_idx(1, 1)
        wait_idx(1)
        fire_gathers(1)
        wait_gathers(0)
        fire_out(0, 0)
        fire_idx(0, 2)

        def outer(g, carry):
            for b in range(2):
                t = 2 * g + 2 + b
                wait_out(b)
                wait_idx(b)
                fire_gathers(b)
                wait_gathers(1 - b)
                fire_out(1 - b, t - 1)
                fire_idx(1 - b, t + 1)
            return carry

        n_pairs = (steps - 3) // 2
        lax.fori_loop(0, n_pairs, outer, 0)

        for t in range(2 * n_pairs + 2, steps):
            b = t % 2
            wait_out(b)
            wait_idx(b)
            fire_gathers(b)
            wait_gathers(1 - b)
            fire_out(1 - b, t - 1)
            if t + 1 < steps:
                fire_idx(1 - b, t + 1)

        last = (steps - 1) % 2
        wait_gathers(last)
        fire_out(last, steps - 1)
        wait_out(1 - last)
        wait_out(last)

    return body


def kernel(input, table):
    s0, s1, s2 = input.shape
    N = s0 * s1 * s2
    idx2d = input.reshape(N // 128, 128).astype(jnp.int32)
    out = _emb_kernel(N)(table, idx2d)
    return out.reshape(s0, s1, s2, D)
